# baseline (speedup 1.0000x reference)
import functools

import jax
import jax.numpy as jnp
from jax import lax
from jax.experimental import pallas as pl
from jax.experimental.pallas import tpu as pltpu
from jax.experimental.pallas import tpu_sc as plsc

_N = 10000
_E = 320000
_D = 128

_NC = 2
_NS = 16
_NW = _NC * _NS
_WIN = 128
_WPT = 80
_EPW = _WPT * _WIN
_E_PAD = _NW * _EPW
_N_PAD = 10112
_RPT = _N_PAD // _NS
_PACK = 16384


def _sc_scatter_add(h, packed2d, zrows):
    mesh = plsc.VectorSubcoreMesh(
        core_axis_name="c", subcore_axis_name="s",
        num_cores=_NC, num_subcores=_NS)

    @functools.partial(
        pl.kernel,
        out_type=jax.ShapeDtypeStruct((_NC, _N_PAD, _D), jnp.float32),
        mesh=mesh,
        scratch_types=[
            pltpu.VMEM((_WPT, _WIN), jnp.int32),
            pltpu.VMEM((2, _WIN), jnp.int32),
            pltpu.VMEM((2, _WIN), jnp.int32),
            pltpu.VMEM((2, _WIN, _D), jnp.float32),
            pltpu.VMEM_SHARED((_N_PAD, _D), jnp.float32),
            pltpu.SemaphoreType.DMA,
            pltpu.SemaphoreType.DMA,
            pltpu.SemaphoreType.DMA,
        ],
    )
    def k(h_hbm, pk_hbm, z_hbm, out_hbm, pk_v, sring, dring, rows_v,
          agg_sh, isem, g0, g1):
        c = lax.axis_index("c")
        s = lax.axis_index("s")
        w = c * _NS + s
        gsem = (g0, g1)

        dstage = pltpu.async_copy(pk_hbm.at[pl.ds(w * _WPT, _WPT)],
                                  pk_v, isem)
        dstage.wait()

        def unpack(i, r):
            for j in range(_WIN // 16):
                v = pk_v[i, pl.ds(j * 16, 16)]
                sring[r, pl.ds(j * 16, 16)] = lax.shift_right_logical(
                    v, 14)
                dring[r, pl.ds(j * 16, 16)] = lax.bitwise_and(
                    v, _PACK - 1)

        def g_start(b):
            pltpu.make_async_copy(h_hbm.at[sring.at[b]], rows_v.at[b],
                                  gsem[b]).start()

        def g_wait(b):
            pltpu.make_async_copy(h_hbm.at[sring.at[b]], rows_v.at[b],
                                  gsem[b]).wait()

        unpack(0, 0)
        g_start(0)
        unpack(1, 1)
        g_start(1)
        pltpu.sync_copy(z_hbm, agg_sh.at[pl.ds(s * _RPT, _RPT)])
        plsc.subcore_barrier()

        def pair(t, carry):
            for b in range(2):
                i = 2 * t + b
                g_wait(b)
                pltpu.sync_copy(rows_v.at[b], agg_sh.at[dring.at[b]],
                                add=True)

                @pl.when(i + 2 < _WPT)
                def _():
                    unpack(i + 2, b)
                    g_start(b)
            return carry

        lax.fori_loop(0, _WPT // 2, pair, 0)

        plsc.subcore_barrier()
        pltpu.sync_copy(agg_sh.at[pl.ds(s * _RPT, _RPT)],
                        out_hbm.at[c, pl.ds(s * _RPT, _RPT)])

    return k(h, packed2d, zrows)


_BT = 2000
_NBLK = _N // _BT


def _zin(z, w1_ref, b1_ref, w2_ref, b2_ref):
    zz = jnp.maximum(
        jnp.dot(z, w1_ref[...], preferred_element_type=jnp.float32)
        + b1_ref[...], 0.0)
    return (jnp.dot(zz, w2_ref[...], preferred_element_type=jnp.float32)
            + b2_ref[...])


def _mlp_body(scale_ref, h_ref, agg_ref, w1_ref, b1_ref, w2_ref, b2_ref,
              out_ref):
    z = h_ref[...] * scale_ref[0, 0] + agg_ref[0] + agg_ref[1]
    out_ref[...] = _zin(z, w1_ref, b1_ref, w2_ref, b2_ref)


def _mlp_head_body(scale_ref, h_ref, agg_ref, w1_ref, b1_ref,
                   w2_ref, b2_ref, lw_ref, lb_ref, fw_ref, fb_ref,
                   out_ref, pool_ref):
    z = h_ref[...] * scale_ref[0, 0] + agg_ref[0] + agg_ref[1]
    o = _zin(z, w1_ref, b1_ref, w2_ref, b2_ref)

    @pl.when(pl.program_id(0) == 0)
    def _():
        pool_ref[...] = jnp.zeros_like(pool_ref)

    pool_ref[...] += jnp.sum(o, axis=0, keepdims=True)

    @pl.when(pl.program_id(0) == _NBLK - 1)
    def _():
        t = jnp.maximum(
            jnp.dot(pool_ref[...], lw_ref[...],
                    preferred_element_type=jnp.float32) + lb_ref[...], 0.0)
        out_ref[...] = (jnp.dot(t, fw_ref[...],
                                preferred_element_type=jnp.float32)
                        + fb_ref[...])


_MLP_SPECS = [
    pl.BlockSpec(memory_space=pltpu.SMEM),
    pl.BlockSpec((_BT, _D), lambda i: (i, 0)),
    pl.BlockSpec((2, _BT, _D), lambda i: (0, i, 0)),
    pl.BlockSpec((_D, _D), lambda i: (0, 0)),
    pl.BlockSpec((1, _D), lambda i: (0, 0)),
    pl.BlockSpec((_D, _D), lambda i: (0, 0)),
    pl.BlockSpec((1, _D), lambda i: (0, 0)),
]


def _tc_mlp(scale, h, agg, w1, b1, w2, b2):
    return pl.pallas_call(
        _mlp_body, grid=(_NBLK,), in_specs=_MLP_SPECS,
        out_specs=pl.BlockSpec((_BT, _D), lambda i: (i, 0)),
        out_shape=jax.ShapeDtypeStruct((_N, _D), jnp.float32),
    )(scale, h, agg, w1, b1, w2, b2)


def _tc_mlp_head(scale, h, agg, w1, b1, w2, b2, lin_w, lin_b, fw_pad,
                 fb_pad):
    head_specs = [pl.BlockSpec((_D, _D), lambda i: (0, 0)),
                  pl.BlockSpec((1, _D), lambda i: (0, 0)),
                  pl.BlockSpec((_D, _D), lambda i: (0, 0)),
                  pl.BlockSpec((1, _D), lambda i: (0, 0))]
    return pl.pallas_call(
        _mlp_head_body, grid=(_NBLK,), in_specs=_MLP_SPECS + head_specs,
        out_specs=pl.BlockSpec((1, _D), lambda i: (0, 0)),
        out_shape=jax.ShapeDtypeStruct((1, _D), jnp.float32),
        scratch_shapes=[pltpu.VMEM((1, _D), jnp.float32)],
    )(scale, h, agg, w1, b1, w2, b2, lin_w, lin_b, fw_pad, fb_pad)


def kernel(x, edge_index, eps0, eps1, eps2,
           W1_0, b1_0, W2_0, b2_0,
           W1_1, b1_1, W2_1, b2_1,
           W1_2, b1_2, W2_2, b2_2,
           lin_W, lin_b, final_W, final_b):
    src = edge_index[0]
    dst = edge_index[1]

    npad = _E_PAD - _E
    pad_ar = jnp.arange(npad, dtype=jnp.int32)
    src_p = jnp.concatenate([src, pad_ar % jnp.int32(_N)])
    dst_p = jnp.concatenate([dst, jnp.int32(_N) + pad_ar % jnp.int32(_N_PAD - _N)])
    packed2d = (src_p * jnp.int32(_PACK) + dst_p).reshape(
        _E_PAD // _WIN, _WIN)
    zrows = jnp.zeros((_RPT, _D), jnp.float32)

    scales = [(1.0 + eps0).reshape(1, 1), (1.0 + eps1).reshape(1, 1),
              (1.0 + eps2).reshape(1, 1)]
    params = [(W1_0, b1_0.reshape(1, _D), W2_0, b2_0.reshape(1, _D)),
              (W1_1, b1_1.reshape(1, _D), W2_1, b2_1.reshape(1, _D)),
              (W1_2, b1_2.reshape(1, _D), W2_2, b2_2.reshape(1, _D))]

    fw_pad = jnp.pad(final_W, ((0, 0), (0, _D - final_W.shape[1])))
    fb_pad = jnp.pad(final_b, (0, _D - final_b.shape[0])).reshape(1, _D)

    h = x
    for l in range(2):
        agg = _sc_scatter_add(h, packed2d, zrows)
        w1, b1, w2, b2 = params[l]
        h = _tc_mlp(scales[l], h, agg, w1, b1, w2, b2)
    agg = _sc_scatter_add(h, packed2d, zrows)
    w1, b1, w2, b2 = params[2]
    out = _tc_mlp_head(scales[2], h, agg, w1, b1, w2, b2,
                       lin_W, lin_b.reshape(1, _D), fw_pad, fb_pad)
    return out[:, :2]
